# Initial kernel scaffold; baseline (speedup 1.0000x reference)
import jax
import jax.numpy as jnp
from jax.experimental import pallas as pl


def kernel(predict_mask, gt_mask):
    raise NotImplementedError("write your pallas kernel here")


# speedup vs baseline: 37.7775x; 37.7775x over previous
import functools

import jax
import jax.numpy as jnp
from jax import lax
from jax.experimental import pallas as pl
from jax.experimental.pallas import tpu as pltpu
from jax.experimental.pallas import tpu_sc as plsc

M = 16384
H2 = 2 * M
CPC = 512 * 512
CH = 8192
NCHUNK = CPC // CH
N_TOT = 4 * CPC


def _sc_lovasz(pred2d, gt2d):
    mesh = plsc.VectorSubcoreMesh(core_axis_name="c", subcore_axis_name="s")

    @functools.partial(
        pl.kernel,
        mesh=mesh,
        compiler_params=pltpu.CompilerParams(needs_layout_passes=False),
        out_type=jax.ShapeDtypeStruct((8, 16), jnp.float32),
        scratch_types=[
            pltpu.VMEM((CH,), jnp.float32),
            pltpu.VMEM((CH,), jnp.float32),
            pltpu.VMEM((CH,), jnp.float32),
            pltpu.VMEM((CH,), jnp.float32),
            pltpu.VMEM((H2,), jnp.float32),
            pltpu.VMEM((16,), jnp.float32),
            pltpu.VMEM_SHARED((16, H2), jnp.float32),
            pltpu.SemaphoreType.DMA,
            pltpu.SemaphoreType.DMA,
        ],
    )
    def k(pred_hbm, gt_hbm, out_hbm, pa, pb, ga, gb, hist, loss_buf,
          shared, sem_a, sem_b):
        cid = lax.axis_index("c")
        sid = lax.axis_index("s")
        ch_local = sid // 4
        ch = cid * 4 + ch_local
        part = sid % 4
        row = part * 8 + ch

        zero16 = jnp.zeros((16,), jnp.float32)
        ones16 = jnp.ones((16,), jnp.float32)

        def zbody(i, c):
            hist[pl.ds(i * 16, 16)] = zero16
            return c
        lax.fori_loop(0, H2 // 16, zbody, 0)

        def issue(kc):
            pbuf, gbuf, sem = (pa, ga, sem_a) if kc % 2 == 0 else (pb, gb, sem_b)
            off = kc * CH
            c1 = pltpu.async_copy(pred_hbm.at[row, pl.ds(off, CH)], pbuf, sem)
            c2 = pltpu.async_copy(gt_hbm.at[row, pl.ds(off, CH)], gbuf, sem)
            return c1, c2

        def make_body(pbuf, gbuf):
            def body(j, c):
                base = j * 16
                p = pbuf[pl.ds(base, 16)]
                g = gbuf[pl.ds(base, 16)]
                err = jnp.abs(g - p)
                bi = jnp.minimum((err * float(M)).astype(jnp.int32), M - 1)
                idx = bi + g.astype(jnp.int32) * M
                plsc.addupdate_scatter(hist, [idx], ones16)
                return c
            return body

        pend = issue(0)
        for kc in range(NCHUNK):
            nxt = issue(kc + 1) if kc + 1 < NCHUNK else None
            pend[0].wait()
            pend[1].wait()
            pbuf, gbuf = (pa, ga) if kc % 2 == 0 else (pb, gb)
            lax.fori_loop(0, CH // 16, make_body(pbuf, gbuf), 0)
            pend = nxt

        pltpu.sync_copy(hist, shared.at[sid])
        plsc.subcore_barrier()

        @pl.when(part == 0)
        def _():
            def addbody(j, c):
                o = j * 16
                hist[pl.ds(c + o, 16)] = hist[pl.ds(c + o, 16)] + pa[pl.ds(o, 16)]
                return c

            for t in range(1, 4):
                for q in range(H2 // CH):
                    pltpu.sync_copy(shared.at[sid + t, pl.ds(q * CH, CH)], pa)
                    lax.fori_loop(0, CH // 16,
                                  lambda j, c: addbody(j, c), q * CH)

            nf = jnp.float32(N_TOT)

            def ph2(t16, carry):
                ctot, cgt, jvec = carry
                o = t16 * 16
                h0 = hist[pl.ds(o, 16)]
                h1 = hist[pl.ds(M + o, 16)]
                ht = h0 + h1
                exc_t = jnp.cumsum(ht) - ht + ctot
                exc_g = jnp.cumsum(h1) - h1 + cgt
                kk = nf - exc_t
                denom = jnp.maximum(kk + exc_g, 1.0)
                jvec = jvec + kk / denom
                return ctot + jnp.sum(ht), cgt + jnp.sum(h1), jvec

            ctot, cgt, jvec = lax.fori_loop(
                0, M // 16, ph2,
                (jnp.float32(0.0), jnp.float32(0.0), zero16))
            h = 1.0 / float(M)
            loss = h * (jnp.sum(jvec) - 1.0) + 0.5 * h
            loss_buf[...] = jnp.full((16,), loss, jnp.float32)
            pltpu.sync_copy(loss_buf, out_hbm.at[ch])

    return k(pred2d, gt2d)


def kernel(predict_mask, gt_mask):
    B, C, H, W = gt_mask.shape
    pred = predict_mask[:, :, :H, :W].reshape(B * C, H * W)
    gt = gt_mask.reshape(B * C, H * W)
    out = _sc_lovasz(pred, gt)
    return out[:, 0].reshape(1, C)


# speedup vs baseline: 41.3936x; 1.0957x over previous
import functools

import jax
import jax.numpy as jnp
from jax import lax
from jax.experimental import pallas as pl
from jax.experimental.pallas import tpu as pltpu
from jax.experimental.pallas import tpu_sc as plsc

M = 16384
HR = (2 * M) // 128
CPC = 512 * 512
CH = 8192
NCHUNK = CPC // CH
N_TOT = 4 * CPC
UNROLL = 8


def _sc_lovasz(pred2d, gt2d):
    mesh = plsc.VectorSubcoreMesh(core_axis_name="c", subcore_axis_name="s")

    @functools.partial(
        pl.kernel,
        mesh=mesh,
        compiler_params=pltpu.CompilerParams(needs_layout_passes=False),
        out_type=jax.ShapeDtypeStruct((8, 16), jnp.float32),
        scratch_types=[
            pltpu.VMEM((CH,), jnp.float32),
            pltpu.VMEM((CH,), jnp.float32),
            pltpu.VMEM((CH,), jnp.float32),
            pltpu.VMEM((CH,), jnp.float32),
            pltpu.VMEM((HR, 128), jnp.float32),
            pltpu.VMEM((16,), jnp.float32),
            pltpu.VMEM((128,), jnp.int32),
            pltpu.VMEM((128,), jnp.int32),
            pltpu.VMEM_SHARED((4 * HR, 128), jnp.float32),
            pltpu.SemaphoreType.DMA,
            pltpu.SemaphoreType.DMA,
            pltpu.SemaphoreType.DMA,
        ],
    )
    def k(pred_hbm, gt_hbm, out_hbm, pa, pb, ga, gb, hist, loss_buf,
          idx_lo, idx_hi, shared, sem_a, sem_b, sem_m):
        cid = lax.axis_index("c")
        sid = lax.axis_index("s")
        ch_local = sid // 4
        ch = cid * 4 + ch_local
        part = sid % 4
        row = part * 8 + ch

        zero16 = jnp.zeros((16,), jnp.float32)
        ones16 = jnp.ones((16,), jnp.float32)
        lane = lax.iota(jnp.int32, 16)

        def zbody(i, c):
            r = i >> 3
            col = (i & 7) * 16
            hist[r, pl.ds(col, 16)] = zero16
            return c
        lax.fori_loop(0, HR * 8, zbody, 0)

        base_row = ch_local * HR

        def ibody(j, c):
            idx_lo[pl.ds(j * 16, 16)] = lane + (base_row + j * 16)
            idx_hi[pl.ds(j * 16, 16)] = lane + (base_row + 128 + j * 16)
            return c
        lax.fori_loop(0, 8, ibody, 0)

        @pl.when(part == 0)
        def _():
            pltpu.sync_copy(hist, shared.at[pl.ds(base_row, HR)])

        plsc.subcore_barrier()

        def issue(kc):
            pbuf, gbuf, sem = (pa, ga, sem_a) if kc % 2 == 0 else (pb, gb, sem_b)
            off = kc * CH
            c1 = pltpu.async_copy(pred_hbm.at[row, pl.ds(off, CH)], pbuf, sem)
            c2 = pltpu.async_copy(gt_hbm.at[row, pl.ds(off, CH)], gbuf, sem)
            return c1, c2

        def make_body(pbuf, gbuf):
            def body(j, c):
                base = j * (16 * UNROLL)
                for u in range(UNROLL):
                    o = base + u * 16
                    p = pbuf[pl.ds(o, 16)]
                    g = gbuf[pl.ds(o, 16)]
                    err = jnp.abs(g - p)
                    bi = jnp.minimum((err * float(M)).astype(jnp.int32), M - 1)
                    idx = bi + g.astype(jnp.int32) * M
                    plsc.addupdate_scatter(hist, [idx >> 7, idx & 127], ones16)
                return c
            return body

        pend = issue(0)
        for kc in range(NCHUNK):
            nxt = issue(kc + 1) if kc + 1 < NCHUNK else None
            pend[0].wait()
            pend[1].wait()
            pbuf, gbuf = (pa, ga) if kc % 2 == 0 else (pb, gb)
            lax.fori_loop(0, CH // (16 * UNROLL), make_body(pbuf, gbuf), 0)
            pend = nxt

        pltpu.sync_copy(hist.at[pl.ds(0, 128)], shared.at[idx_lo], add=True)
        pltpu.sync_copy(hist.at[pl.ds(128, 128)], shared.at[idx_hi], add=True)
        plsc.subcore_barrier()

        @pl.when(part == 0)
        def _():
            pltpu.sync_copy(shared.at[pl.ds(base_row, HR)], hist)

            nf = jnp.float32(N_TOT)

            def ph2(t16, carry):
                ctot, cgt, jvec = carry
                for u in range(2):
                    i = t16 * 2 + u
                    r = i >> 3
                    col = (i & 7) * 16
                    h0 = hist[r, pl.ds(col, 16)]
                    h1 = hist[r + HR // 2, pl.ds(col, 16)]
                    ht = h0 + h1
                    inc_t = jnp.cumsum(ht)
                    inc_g = jnp.cumsum(h1)
                    kk = nf - (inc_t - ht + ctot)
                    denom = jnp.maximum(kk + (inc_g - h1 + cgt), 1.0)
                    jvec = jvec + kk / denom
                    ctot = ctot + inc_t[15]
                    cgt = cgt + inc_g[15]
                return ctot, cgt, jvec

            ctot, cgt, jvec = lax.fori_loop(
                0, M // 32, ph2,
                (jnp.float32(0.0), jnp.float32(0.0), zero16))
            h = 1.0 / float(M)
            loss = h * (jnp.sum(jvec) - 1.0) + 0.5 * h
            loss_buf[...] = jnp.full((16,), loss, jnp.float32)
            pltpu.sync_copy(loss_buf, out_hbm.at[ch])

    return k(pred2d, gt2d)


def kernel(predict_mask, gt_mask):
    B, C, H, W = gt_mask.shape
    pred = predict_mask[:, :, :H, :W].reshape(B * C, H * W)
    gt = gt_mask.reshape(B * C, H * W)
    out = _sc_lovasz(pred, gt)
    return out[:, 0].reshape(1, C)


# speedup vs baseline: 99.1029x; 2.3942x over previous
import functools

import jax
import jax.numpy as jnp
from jax import lax
from jax.experimental import pallas as pl
from jax.experimental.pallas import tpu as pltpu
from jax.experimental.pallas import tpu_sc as plsc

M = 16384
HR = (2 * M) // 128
CPC = 512 * 512
CH = 8192
NCHUNK = CPC // CH
N_TOT = 4 * CPC
UNROLL = 8


def _sc_lovasz(pred2d, gt2d):
    mesh = plsc.VectorSubcoreMesh(core_axis_name="c", subcore_axis_name="s")

    @functools.partial(
        pl.kernel,
        mesh=mesh,
        compiler_params=pltpu.CompilerParams(needs_layout_passes=False),
        out_type=jax.ShapeDtypeStruct((8, 16), jnp.float32),
        scratch_types=[
            pltpu.VMEM((CH,), jnp.float32),
            pltpu.VMEM((CH,), jnp.float32),
            pltpu.VMEM((CH,), jnp.float32),
            pltpu.VMEM((CH,), jnp.float32),
            pltpu.VMEM((HR, 128), jnp.float32),
            pltpu.VMEM((16,), jnp.float32),
            pltpu.VMEM((128,), jnp.int32),
            pltpu.VMEM((128,), jnp.int32),
            pltpu.VMEM_SHARED((4 * HR, 128), jnp.float32),
            pltpu.SemaphoreType.DMA,
            pltpu.SemaphoreType.DMA,
            pltpu.SemaphoreType.DMA,
        ],
    )
    def k(pred_hbm, gt_hbm, out_hbm, pa, pb, ga, gb, hist, loss_buf,
          idx_lo, idx_hi, shared, sem_a, sem_b, sem_m):
        cid = lax.axis_index("c")
        sid = lax.axis_index("s")
        ch_local = sid // 4
        ch = cid * 4 + ch_local
        part = sid % 4
        row = part * 8 + ch

        zero16 = jnp.zeros((16,), jnp.float32)
        ones16 = jnp.ones((16,), jnp.float32)
        lane = lax.iota(jnp.int32, 16)

        @plsc.parallel_loop(0, HR * 8, unroll=8)
        def _(i):
            r = i >> 3
            col = (i & 7) * 16
            hist[r, pl.ds(col, 16)] = zero16

        base_row = ch_local * HR

        def ibody(j, c):
            idx_lo[pl.ds(j * 16, 16)] = lane + (base_row + j * 16)
            idx_hi[pl.ds(j * 16, 16)] = lane + (base_row + 128 + j * 16)
            return c
        lax.fori_loop(0, 8, ibody, 0)

        @pl.when(part == 0)
        def _():
            pltpu.sync_copy(hist, shared.at[pl.ds(base_row, HR)])

        plsc.subcore_barrier()

        def issue(kc):
            pbuf, gbuf, sem = (pa, ga, sem_a) if kc % 2 == 0 else (pb, gb, sem_b)
            off = kc * CH
            c1 = pltpu.async_copy(pred_hbm.at[row, pl.ds(off, CH)], pbuf, sem)
            c2 = pltpu.async_copy(gt_hbm.at[row, pl.ds(off, CH)], gbuf, sem)
            return c1, c2

        def process(pbuf, gbuf):
            @plsc.parallel_loop(0, CH // 16, unroll=UNROLL)
            def _(j):
                o = j * 16
                p = pbuf[pl.ds(o, 16)]
                g = gbuf[pl.ds(o, 16)]
                err = jnp.abs(g - p)
                bi = jnp.minimum((err * float(M)).astype(jnp.int32), M - 1)
                idx = bi + g.astype(jnp.int32) * M
                plsc.addupdate_scatter(hist, [idx >> 7, idx & 127], ones16)

        pend = issue(0)
        for kc in range(NCHUNK):
            nxt = issue(kc + 1) if kc + 1 < NCHUNK else None
            pend[0].wait()
            pend[1].wait()
            pbuf, gbuf = (pa, ga) if kc % 2 == 0 else (pb, gb)
            process(pbuf, gbuf)
            pend = nxt

        pltpu.sync_copy(hist.at[pl.ds(0, 128)], shared.at[idx_lo], add=True)
        pltpu.sync_copy(hist.at[pl.ds(128, 128)], shared.at[idx_hi], add=True)
        plsc.subcore_barrier()

        @pl.when(part == 0)
        def _():
            pltpu.sync_copy(shared.at[pl.ds(base_row, HR)], hist)

            nf = jnp.float32(N_TOT)

            @plsc.parallel_loop(
                0, M // 16, unroll=4,
                carry=(jnp.float32(0.0), jnp.float32(0.0), zero16))
            def ph2(i, carry):
                ctot, cgt, jvec = carry
                r = i >> 3
                col = (i & 7) * 16
                h0 = hist[r, pl.ds(col, 16)]
                h1 = hist[r + HR // 2, pl.ds(col, 16)]
                ht = h0 + h1
                inc_t = jnp.cumsum(ht)
                inc_g = jnp.cumsum(h1)
                kk = nf - (inc_t - ht + ctot)
                denom = jnp.maximum(kk + (inc_g - h1 + cgt), 1.0)
                jvec = jvec + kk / denom
                return ctot + inc_t[15], cgt + inc_g[15], jvec

            ctot, cgt, jvec = ph2
            h = 1.0 / float(M)
            loss = h * (jnp.sum(jvec) - 1.0) + 0.5 * h
            loss_buf[...] = jnp.full((16,), loss, jnp.float32)
            pltpu.sync_copy(loss_buf, out_hbm.at[ch])

    return k(pred2d, gt2d)


def kernel(predict_mask, gt_mask):
    B, C, H, W = gt_mask.shape
    pred = predict_mask[:, :, :H, :W].reshape(B * C, H * W)
    gt = gt_mask.reshape(B * C, H * W)
    out = _sc_lovasz(pred, gt)
    return out[:, 0].reshape(1, C)


# speedup vs baseline: 172.6225x; 1.7419x over previous
import functools

import jax
import jax.numpy as jnp
from jax import lax
from jax.experimental import pallas as pl
from jax.experimental.pallas import tpu as pltpu
from jax.experimental.pallas import tpu_sc as plsc

M = 16384
HR = (2 * M) // 128
ROWS_C = 16
CH = ROWS_C * 512
NCHUNK = 512 // ROWS_C
N_TOT = 4 * 512 * 512
UNROLL = 8


def _sc_lovasz(pred4d, gt4d):
    mesh = plsc.VectorSubcoreMesh(core_axis_name="c", subcore_axis_name="s")

    @functools.partial(
        pl.kernel,
        mesh=mesh,
        compiler_params=pltpu.CompilerParams(
            needs_layout_passes=False, use_tc_tiling_on_sc=True),
        out_type=jax.ShapeDtypeStruct((8, 16), jnp.float32),
        scratch_types=[
            pltpu.VMEM((ROWS_C, 512), jnp.float32),
            pltpu.VMEM((ROWS_C, 512), jnp.float32),
            pltpu.VMEM((ROWS_C, 512), jnp.float32),
            pltpu.VMEM((ROWS_C, 512), jnp.float32),
            pltpu.VMEM((HR, 128), jnp.float32),
            pltpu.VMEM((16,), jnp.float32),
            pltpu.VMEM((128,), jnp.int32),
            pltpu.VMEM((128,), jnp.int32),
            pltpu.VMEM_SHARED((4 * HR, 128), jnp.float32),
            pltpu.SemaphoreType.DMA,
            pltpu.SemaphoreType.DMA,
            pltpu.SemaphoreType.DMA,
        ],
    )
    def k(pred_hbm, gt_hbm, out_hbm, pa, pb, ga, gb, hist, loss_buf,
          idx_lo, idx_hi, shared, sem_a, sem_b, sem_m):
        cid = lax.axis_index("c")
        sid = lax.axis_index("s")
        ch_local = sid // 4
        ch = cid * 4 + ch_local
        part = sid % 4

        zero16 = jnp.zeros((16,), jnp.float32)
        ones16 = jnp.ones((16,), jnp.float32)
        clamp16 = jnp.full((16,), 2 * M - 1, jnp.int32)
        lane = lax.iota(jnp.int32, 16)

        @plsc.parallel_loop(0, HR * 8, unroll=8)
        def _(i):
            r = i >> 3
            col = (i & 7) * 16
            hist[r, pl.ds(col, 16)] = zero16

        base_row = ch_local * HR

        def ibody(j, c):
            idx_lo[pl.ds(j * 16, 16)] = lane + (base_row + j * 16)
            idx_hi[pl.ds(j * 16, 16)] = lane + (base_row + 128 + j * 16)
            return c
        lax.fori_loop(0, 8, ibody, 0)

        @pl.when(part == 0)
        def _():
            pltpu.sync_copy(hist, shared.at[pl.ds(base_row, HR)])

        plsc.subcore_barrier()

        def issue(kc):
            pbuf, gbuf, sem = (pa, ga, sem_a) if kc % 2 == 0 else (pb, gb, sem_b)
            r0 = kc * ROWS_C
            c1 = pltpu.async_copy(
                pred_hbm.at[part, ch, pl.ds(r0, ROWS_C), :], pbuf, sem)
            c2 = pltpu.async_copy(
                gt_hbm.at[part, ch, pl.ds(r0, ROWS_C), :], gbuf, sem)
            return c1, c2

        fm = jnp.float32(M)

        def process(pbuf, gbuf):
            @plsc.parallel_loop(0, CH // 16, unroll=UNROLL)
            def _(j):
                r = j >> 5
                col = (j & 31) * 16
                p = pbuf[r, pl.ds(col, 16)]
                g = gbuf[r, pl.ds(col, 16)]
                key = jnp.abs(g - p) + g
                idx = jnp.minimum((key * fm).astype(jnp.int32), clamp16)
                plsc.addupdate_scatter(hist, [idx >> 7, idx & 127], ones16)

        pend = issue(0)
        for kc in range(NCHUNK):
            nxt = issue(kc + 1) if kc + 1 < NCHUNK else None
            pend[0].wait()
            pend[1].wait()
            pbuf, gbuf = (pa, ga) if kc % 2 == 0 else (pb, gb)
            process(pbuf, gbuf)
            pend = nxt

        pltpu.sync_copy(hist.at[pl.ds(0, 128)], shared.at[idx_lo], add=True)
        pltpu.sync_copy(hist.at[pl.ds(128, 128)], shared.at[idx_hi], add=True)
        plsc.subcore_barrier()

        @pl.when(part == 0)
        def _():
            pltpu.sync_copy(shared.at[pl.ds(base_row, HR)], hist)

            nf = jnp.float32(N_TOT)

            @plsc.parallel_loop(
                0, M // 16, unroll=4,
                carry=(jnp.float32(0.0), jnp.float32(0.0), zero16))
            def ph2(i, carry):
                ctot, cgt, jvec = carry
                r = i >> 3
                col = (i & 7) * 16
                h0 = hist[r, pl.ds(col, 16)]
                h1 = hist[r + HR // 2, pl.ds(col, 16)]
                ht = h0 + h1
                inc_t = jnp.cumsum(ht)
                inc_g = jnp.cumsum(h1)
                kk = nf - (inc_t - ht + ctot)
                denom = jnp.maximum(kk + (inc_g - h1 + cgt), 1.0)
                jvec = jvec + kk / denom
                return ctot + inc_t[15], cgt + inc_g[15], jvec

            ctot, cgt, jvec = ph2
            h = 1.0 / float(M)
            loss = h * (jnp.sum(jvec) - 1.0) + 0.5 * h
            loss_buf[...] = jnp.full((16,), loss, jnp.float32)
            pltpu.sync_copy(loss_buf, out_hbm.at[ch])

    return k(pred4d, gt4d)


def kernel(predict_mask, gt_mask):
    B, C, H, W = gt_mask.shape
    pred = predict_mask[:, :, :H, :W]
    out = _sc_lovasz(pred, gt_mask)
    return out[:, 0].reshape(1, C)
